# baseline (speedup 1.0000x reference)
import jax
import jax.numpy as jnp
from jax.experimental import pallas as pl
from jax.experimental.pallas import tpu as pltpu

_B = 64

_C1_W = ((0, 85), (72, 80))
_C1_WO = (80, 76)
_C1_J = (40, 38)
_C2_WO = (36, 34)
_C2_J = (18, 17)


def _rshift(v, s):
    if s == 0:
        return v
    return jnp.concatenate(
        [v[s:], jnp.zeros((s,) + v.shape[1:], v.dtype)], axis=0)


def _lshift1(v):
    return jnp.concatenate(
        [v[:, 1:], jnp.zeros((v.shape[0], 1), v.dtype)], axis=1)


def _net_kernel(x_ref,
                t1a_ref, t1b_ref, b1a_ref, b1b_ref, s1a_ref, s1b_ref,
                t2a_ref, t2b_ref, b2a_ref, b2b_ref, s2a_ref, s2b_ref,
                e2_ref, w1_ref, f1b_ref, w2_ref, f2b_ref, w3_ref, f3b_ref,
                o_ref):
    B = x_ref.shape[0]
    R = B * 12
    f32 = jnp.float32
    bf16 = jnp.bfloat16
    t1 = (t1a_ref, t1b_ref)
    s1 = (s1a_ref, s1b_ref)
    b1 = (b1a_ref, b1b_ref)
    t2 = (t2a_ref, t2b_ref)
    s2 = (s2a_ref, s2b_ref)
    b2 = (b2a_ref, b2b_ref)

    xc = []
    vall = x_ref[...].astype(bf16)
    for p in range(4):
        vci = [vall[:, ci, :, p * 152:(p + 1) * 152].reshape(R, 152)
               for ci in range(3)]
        xc.append([
            jnp.concatenate([v[:, w0:w0 + wd] for v in vci], axis=1)
            for (w0, wd) in _C1_W])
    xs1 = [[c, _rshift(c, 1)] for c in
           (xc[0][0], xc[1][0], xc[2][0], xc[3][0])]
    xs2 = [[c, _rshift(c, 1)] for c in
           (xc[0][1], xc[1][1], xc[2][1], xc[3][1])]

    x2p = [[None, None], [None, None]]
    for c, xsc in ((0, xs1), (1, xs2)):
        ncol = 6 * _C1_WO[c]
        acc = [jnp.zeros((R, ncol), f32) for _ in range(4)]
        for dy in range(5):
            for p in range(4):
                q, sh = (p + dy) % 4, (p + dy) // 4
                acc[p] = acc[p] + jnp.dot(xsc[q][sh], t1[c][dy],
                                          preferred_element_type=f32)
        acc = [jnp.maximum(s + b1[c][...], 0.0).astype(bf16) for s in acc]
        for e in range(2):
            cm = jnp.maximum(acc[2 * e], acc[2 * e + 1])
            cm = jnp.maximum(cm, _lshift1(cm))
            x2p[e][c] = jnp.dot(cm, s1[c][...],
                                preferred_element_type=f32).astype(bf16)

    x2s = [[[v, _rshift(v, 1), _rshift(v, 2)] for v in (x2p[0][c], x2p[1][c])]
           for c in range(2)]
    p2 = [None, None]
    for c in range(2):
        ncol = 16 * _C2_WO[c]
        a2 = [jnp.zeros((R, ncol), f32) for _ in range(2)]
        for dy in range(5):
            for e in range(2):
                q, sh = (e + dy) % 2, (e + dy) // 2
                a2[e] = a2[e] + jnp.dot(x2s[c][q][sh], t2[c][dy],
                                        preferred_element_type=f32)
        a2 = [jnp.maximum(s + b2[c][...], 0.0) for s in a2]
        fr = jnp.maximum(a2[0], a2[1]).astype(bf16)
        cm2 = jnp.maximum(fr, _lshift1(fr))
        p2[c] = jnp.dot(cm2, s2[c][...],
                        preferred_element_type=f32).astype(bf16)
    p2cat = jnp.concatenate(p2, axis=1)

    h = jnp.zeros((B, 120), f32)
    for k in range(9):
        fk = jnp.dot(e2_ref[k], p2cat, preferred_element_type=f32).astype(bf16)
        h = h + jnp.dot(fk, w1_ref[k], preferred_element_type=f32)
    h = jnp.maximum(h + f1b_ref[...], 0.0).astype(bf16)
    h = jnp.dot(h, w2_ref[...], preferred_element_type=f32) + f2b_ref[...]
    h = jnp.maximum(h, 0.0).astype(bf16)
    o_ref[...] = jnp.dot(h, w3_ref[...], preferred_element_type=f32) + f3b_ref[...]


def _toeplitz(wconv, wsrc, wout):
    kw = wconv.shape[3]
    w = jnp.arange(wsrc)
    wo = jnp.arange(wout)
    dx = jnp.arange(kw)
    mask = (w[None, :, None] == wo[None, None, :] + dx[:, None, None])
    t = jnp.einsum('xwv,ocdx->dcwov', mask.astype(jnp.float32), wconv)
    kh, cin, cout = wconv.shape[2], wconv.shape[1], wconv.shape[0]
    return t.reshape(kh, cin * wsrc, cout * wout).astype(jnp.bfloat16)


def _even_col_selector(nch, wout, wp):
    r = jnp.arange(nch * wout)
    c = jnp.arange(nch * wp)
    sel = (r[:, None] // wout == c[None, :] // wp) & (
        r[:, None] % wout == 2 * (c[None, :] % wp))
    return sel.astype(jnp.bfloat16)


def kernel(c1w, c1b, c2w, c2b, f1w, f1b, f2w, f2b, f3w, f3b, x):
    N = x.shape[0]
    B = _B
    f32 = jnp.float32
    bf16 = jnp.bfloat16

    xr = x.astype(f32).reshape(N, 3, 12, 4 * 152)

    w1c = c1w.reshape(6, 3, 5, 5)
    w2c = c2w.reshape(16, 6, 5, 5)
    t1a = _toeplitz(w1c, _C1_W[0][1], _C1_WO[0])
    t1b = _toeplitz(w1c, _C1_W[1][1], _C1_WO[1])
    t2a = _toeplitz(w2c, _C1_J[0], _C2_WO[0])
    t2b = _toeplitz(w2c, _C1_J[1], _C2_WO[1])
    s1a = _even_col_selector(6, _C1_WO[0], _C1_J[0])
    s1b = _even_col_selector(6, _C1_WO[1], _C1_J[1])
    s2a = _even_col_selector(16, _C2_WO[0], _C2_J[0])
    s2b = _even_col_selector(16, _C2_WO[1], _C2_J[1])
    b1a = jnp.repeat(c1b, _C1_WO[0])[None, :].astype(f32)
    b1b = jnp.repeat(c1b, _C1_WO[1])[None, :].astype(f32)
    b2a = jnp.repeat(c2b, _C2_WO[0])[None, :].astype(f32)
    b2b = jnp.repeat(c2b, _C2_WO[1])[None, :].astype(f32)
    kk = jnp.arange(9)
    bb = jnp.arange(B)
    cc = jnp.arange(B * 12)
    e2 = (cc[None, None, :] == bb[None, :, None] * 12
          + kk[:, None, None]).astype(bf16)
    wA = f1w.reshape(16, 9, 35, 120)
    part0 = wA[:, :, 0:18].transpose(1, 0, 2, 3).reshape(9, 16 * 18, 120)
    part1 = wA[:, :, 18:35].transpose(1, 0, 2, 3).reshape(9, 16 * 17, 120)
    w1r = jnp.concatenate([part0, part1], axis=1).astype(bf16)
    w2b = f2w.astype(bf16)
    w3b = f3w.astype(bf16)

    full = lambda arr: pl.BlockSpec(arr.shape, lambda n: (0,) * arr.ndim)
    out = pl.pallas_call(
        _net_kernel,
        out_shape=jax.ShapeDtypeStruct((N, 8), f32),
        grid=(N // B,),
        in_specs=[
            pl.BlockSpec((B, 3, 12, 608), lambda n: (n, 0, 0, 0)),
            full(t1a), full(t1b), full(b1a), full(b1b), full(s1a), full(s1b),
            full(t2a), full(t2b), full(b2a), full(b2b), full(s2a), full(s2b),
            full(e2), full(w1r), full(f1b), full(w2b), full(f2b),
            full(w3b), full(f3b),
        ],
        out_specs=pl.BlockSpec((B, 8), lambda n: (n, 0)),
        compiler_params=pltpu.CompilerParams(
            dimension_semantics=("parallel",)),
    )(xr, t1a, t1b, b1a, b1b, s1a, s1b,
      t2a, t2b, b2a, b2b, s2a, s2b, e2, w1r, f1b, w2b, f2b, w3b, f3b)
    return out


# speedup vs baseline: 1.0059x; 1.0059x over previous
import jax
import jax.numpy as jnp
from jax.experimental import pallas as pl
from jax.experimental.pallas import tpu as pltpu

_B = 64

_C1_W = ((0, 85), (72, 80))
_C1_WO = (80, 76)
_C1_J = (40, 38)
_C2_WO = (36, 34)
_C2_J = (18, 17)


def _rshift(v, s):
    if s == 0:
        return v
    return jnp.concatenate(
        [v[s:], jnp.zeros((s,) + v.shape[1:], v.dtype)], axis=0)


def _lshift1(v):
    return jnp.concatenate(
        [v[:, 1:], jnp.zeros((v.shape[0], 1), v.dtype)], axis=1)


_TSEG = (512, 512, 640, 640)
_TCOL = (480, 456, 576, 544)


def _net_kernel(x_ref, tt_ref, bb_ref, s1a_ref, s1b_ref, s2a_ref, s2b_ref,
                e2_ref, w1_ref, f1b_ref, w2_ref, f2b_ref, w3_ref, f3b_ref,
                o_ref):
    B = x_ref.shape[0]
    R = B * 12
    f32 = jnp.float32
    bf16 = jnp.bfloat16
    o0, o1, o2 = _TSEG[0], _TSEG[0] + _TSEG[1], _TSEG[0] + _TSEG[1] + _TSEG[2]
    t1 = (tt_ref[:, 0:_TCOL[0]], tt_ref[:, o0:o0 + _TCOL[1]])
    t2 = (tt_ref[:, o1:o1 + _TCOL[2]], tt_ref[:, o2:o2 + _TCOL[3]])
    b1 = (bb_ref[:, 0:_TCOL[0]], bb_ref[:, o0:o0 + _TCOL[1]])
    b2 = (bb_ref[:, o1:o1 + _TCOL[2]], bb_ref[:, o2:o2 + _TCOL[3]])
    s1 = (s1a_ref, s1b_ref)
    s2 = (s2a_ref, s2b_ref)

    xc = []
    zpad = [jnp.zeros((R, 256 - 3 * wd), bf16) for (_w0, wd) in _C1_W]
    vall = x_ref[...].astype(bf16)
    for p in range(4):
        vci = [vall[:, ci, :, p * 152:(p + 1) * 152].reshape(R, 152)
               for ci in range(3)]
        xc.append([
            jnp.concatenate([v[:, w0:w0 + wd] for v in vci] + [zpad[ic]],
                            axis=1)
            for ic, (w0, wd) in enumerate(_C1_W)])
    xs1 = [[c, _rshift(c, 1)] for c in
           (xc[0][0], xc[1][0], xc[2][0], xc[3][0])]
    xs2 = [[c, _rshift(c, 1)] for c in
           (xc[0][1], xc[1][1], xc[2][1], xc[3][1])]

    x2p = [[None, None], [None, None]]
    for c, xsc in ((0, xs1), (1, xs2)):
        for e in range(2):
            ab = []
            for p in (2 * e, 2 * e + 1):
                lhs = jnp.concatenate(
                    [xsc[(p + dy) % 4][(p + dy) // 4] for dy in range(5)],
                    axis=1)
                s = jnp.dot(lhs, t1[c], preferred_element_type=f32)
                ab.append(jnp.maximum(s + b1[c], 0.0).astype(bf16))
            cm = jnp.maximum(ab[0], ab[1])
            cm = jnp.maximum(cm, _lshift1(cm))
            x2p[e][c] = jnp.dot(cm, s1[c][...],
                                preferred_element_type=f32).astype(bf16)

    x2s = [[[v, _rshift(v, 1), _rshift(v, 2)] for v in (x2p[0][c], x2p[1][c])]
           for c in range(2)]
    p2 = [None, None]
    for c in range(2):
        a2 = []
        for e in range(2):
            lhs = jnp.concatenate(
                [x2s[c][(e + dy) % 2][(e + dy) // 2] for dy in range(5)],
                axis=1)
            s = jnp.dot(lhs, t2[c], preferred_element_type=f32)
            a2.append(jnp.maximum(s + b2[c], 0.0))
        fr = jnp.maximum(a2[0], a2[1]).astype(bf16)
        cm2 = jnp.maximum(fr, _lshift1(fr))
        p2[c] = jnp.dot(cm2, s2[c][...],
                        preferred_element_type=f32).astype(bf16)
    p2cat = jnp.concatenate(p2, axis=1)

    feats = jnp.dot(e2_ref[...], p2cat,
                    preferred_element_type=f32).astype(bf16)
    h = jnp.zeros((B, 120), f32)
    for k in range(9):
        h = h + jnp.dot(feats[k * B:(k + 1) * B], w1_ref[k],
                        preferred_element_type=f32)
    h = jnp.maximum(h + f1b_ref[...], 0.0).astype(bf16)
    h = jnp.dot(h, w2_ref[...], preferred_element_type=f32) + f2b_ref[...]
    h = jnp.maximum(h, 0.0).astype(bf16)
    o_ref[...] = jnp.dot(h, w3_ref[...], preferred_element_type=f32) + f3b_ref[...]


def _toeplitz(wconv, wsrc, wout):
    kw = wconv.shape[3]
    w = jnp.arange(wsrc)
    wo = jnp.arange(wout)
    dx = jnp.arange(kw)
    mask = (w[None, :, None] == wo[None, None, :] + dx[:, None, None])
    t = jnp.einsum('xwv,ocdx->dcwov', mask.astype(jnp.float32), wconv)
    kh, cin, cout = wconv.shape[2], wconv.shape[1], wconv.shape[0]
    return t.reshape(kh, cin * wsrc, cout * wout).astype(jnp.bfloat16)


def _even_col_selector(nch, wout, wp, pad_to=None):
    r = jnp.arange(nch * wout)
    c = jnp.arange(nch * wp)
    sel = (r[:, None] // wout == c[None, :] // wp) & (
        r[:, None] % wout == 2 * (c[None, :] % wp))
    sel = sel.astype(jnp.bfloat16)
    if pad_to is not None:
        sel = jnp.pad(sel, ((0, 0), (0, pad_to - nch * wp)))
    return sel


def _stack_k(t, kdim):
    return jnp.pad(t, ((0, 0), (0, 256 - kdim), (0, 0))).reshape(
        5 * 256, t.shape[2])


def kernel(c1w, c1b, c2w, c2b, f1w, f1b, f2w, f2b, f3w, f3b, x):
    N = x.shape[0]
    B = _B
    f32 = jnp.float32
    bf16 = jnp.bfloat16

    xr = x.astype(f32).reshape(N, 3, 12, 4 * 152)

    w1c = c1w.reshape(6, 3, 5, 5)
    w2c = c2w.reshape(16, 6, 5, 5)
    t1a = _stack_k(_toeplitz(w1c, _C1_W[0][1], _C1_WO[0]), 3 * _C1_W[0][1])
    t1b = _stack_k(_toeplitz(w1c, _C1_W[1][1], _C1_WO[1]), 3 * _C1_W[1][1])
    t2a = _stack_k(_toeplitz(w2c, _C1_J[0], _C2_WO[0]), 6 * _C1_J[0])
    t2b = _stack_k(_toeplitz(w2c, _C1_J[1], _C2_WO[1]), 6 * _C1_J[1])
    pads = [seg - col for seg, col in zip(_TSEG, _TCOL)]
    tt = jnp.concatenate(
        [jnp.pad(t, ((0, 0), (0, p))) for t, p in
         zip((t1a, t1b, t2a, t2b), pads)], axis=1)
    b1a = jnp.repeat(c1b, _C1_WO[0])[None, :].astype(f32)
    b1b = jnp.repeat(c1b, _C1_WO[1])[None, :].astype(f32)
    b2a = jnp.repeat(c2b, _C2_WO[0])[None, :].astype(f32)
    b2b = jnp.repeat(c2b, _C2_WO[1])[None, :].astype(f32)
    bb = jnp.concatenate(
        [jnp.pad(b, ((0, 0), (0, p))) for b, p in
         zip((b1a, b1b, b2a, b2b), pads)], axis=1)
    s1a = _even_col_selector(6, _C1_WO[0], _C1_J[0], 256)
    s1b = _even_col_selector(6, _C1_WO[1], _C1_J[1], 256)
    s2a = _even_col_selector(16, _C2_WO[0], _C2_J[0])
    s2b = _even_col_selector(16, _C2_WO[1], _C2_J[1])
    rr = jnp.arange(9 * B)
    cc = jnp.arange(B * 12)
    e2 = (cc[None, :] == (rr[:, None] % B) * 12
          + rr[:, None] // B).astype(bf16)
    wA = f1w.reshape(16, 9, 35, 120)
    part0 = wA[:, :, 0:18].transpose(1, 0, 2, 3).reshape(9, 16 * 18, 120)
    part1 = wA[:, :, 18:35].transpose(1, 0, 2, 3).reshape(9, 16 * 17, 120)
    w1r = jnp.concatenate([part0, part1], axis=1).astype(bf16)
    w2b = f2w.astype(bf16)
    w3b = f3w.astype(bf16)

    full = lambda arr: pl.BlockSpec(arr.shape, lambda n: (0,) * arr.ndim)
    out = pl.pallas_call(
        _net_kernel,
        out_shape=jax.ShapeDtypeStruct((N, 8), f32),
        grid=(N // B,),
        in_specs=[
            pl.BlockSpec((B, 3, 12, 608), lambda n: (n, 0, 0, 0)),
            full(tt), full(bb), full(s1a), full(s1b), full(s2a), full(s2b),
            full(e2), full(w1r), full(f1b), full(w2b), full(f2b),
            full(w3b), full(f3b),
        ],
        out_specs=pl.BlockSpec((B, 8), lambda n: (n, 0)),
        compiler_params=pltpu.CompilerParams(
            dimension_semantics=("arbitrary",)),
    )(xr, tt, bb, s1a, s1b, s2a, s2b, e2, w1r, f1b, w2b, f2b, w3b, f3b)
    return out
